# Initial kernel scaffold; baseline (speedup 1.0000x reference)
import jax
import jax.numpy as jnp
from jax.experimental import pallas as pl


def kernel(x, weight):
    raise NotImplementedError("write your pallas kernel here")


# speedup vs baseline: 78.9024x; 78.9024x over previous
import functools

import jax
import jax.numpy as jnp
from jax import lax
from jax.experimental import pallas as pl
from jax.experimental.pallas import tpu as pltpu
from jax.experimental.pallas import tpu_sc as plsc

V = 1000000
D = 64
R = 4096
C = 200
NC = 2
NS = 16
NW = NC * NS
XPW = R // NW
CA = 104
CB = C - CA
NBUF = 4
ROUNDS = XPW // NBUF

_mesh = plsc.VectorSubcoreMesh(core_axis_name="c", subcore_axis_name="s")


@functools.partial(
    pl.kernel,
    mesh=_mesh,
    out_type=jax.ShapeDtypeStruct((R, C, D), jnp.float32),
    compiler_params=pltpu.CompilerParams(use_tc_tiling_on_sc=False),
    scratch_types=[
        pltpu.VMEM((XPW, C), jnp.int32),
        pltpu.VMEM((NBUF, C, D), jnp.float32),
        pltpu.SemaphoreType.DMA((NBUF,)),
    ],
)
def _gather_kernel(x_hbm, table_hbm, out_hbm, idx_v, rows_v, gsem):
    wid = lax.axis_index("s") * NC + lax.axis_index("c")
    xbase = wid * XPW

    pltpu.sync_copy(x_hbm.at[pl.ds(xbase, XPW)], idx_v)

    def gather_descs(j, s):
        a = pltpu.make_async_copy(
            table_hbm.at[idx_v.at[j, pl.ds(0, CA)]],
            rows_v.at[s, pl.ds(0, CA)],
            gsem.at[s],
        )
        b = pltpu.make_async_copy(
            table_hbm.at[idx_v.at[j, pl.ds(CA, CB)]],
            rows_v.at[s, pl.ds(CA, CB)],
            gsem.at[s],
        )
        return a, b

    def start_gathers(j, s):
        a, b = gather_descs(j, s)
        a.start()
        b.start()

    def wait_gathers(j, s):
        a, b = gather_descs(j, s)
        a.wait()
        b.wait()

    def copy_out(j, s):
        pltpu.sync_copy(rows_v.at[s], out_hbm.at[xbase + j])

    for s in range(NBUF):
        start_gathers(s, s)

    def round_body(r, carry):
        for s in range(NBUF):
            j = r * NBUF + s
            wait_gathers(j, s)
            copy_out(j, s)
            start_gathers(j + NBUF, s)
        return carry

    lax.fori_loop(0, ROUNDS - 1, round_body, 0)

    for s in range(NBUF):
        j = (ROUNDS - 1) * NBUF + s
        wait_gathers(j, s)
        copy_out(j, s)


def kernel(x, weight):
    return _gather_kernel(x.astype(jnp.int32), weight)
